# Initial kernel scaffold; baseline (speedup 1.0000x reference)
import jax
import jax.numpy as jnp
from jax.experimental import pallas as pl


def kernel(seq1, seq2, edge_index, loc, eps, W1s, b1s, g1s, be1s, W2s, b2s, g2s, be2s, Wb, bb):
    raise NotImplementedError("write your pallas kernel here")


# speedup vs baseline: 4.6901x; 4.6901x over previous
import functools

import jax
import jax.numpy as jnp
from jax import lax
from jax.experimental import pallas as pl
from jax.experimental.pallas import tpu as pltpu
from jax.experimental.pallas import tpu_sc as plsc

N = 10000
D = 128
E = 320000
K = 10
KP = 128
NLAYERS = 2
NC = 2
NS = 16
EP = E // NS
CW = 80
CH2 = EP // CW + 2
EPP = CH2 * CW
NH = N // 2
AROWS = NH + 8
SRT = 320
BN = 2000
NBLK = N // BN


def _seg_body(h2n, srcs, dsts, zeros, out,
              src_c, dst_c, rows, acc, sem_g, sem_s):
    c = lax.axis_index("c")
    s = lax.axis_index("s")
    pltpu.sync_copy(srcs.at[s], src_c)
    pltpu.sync_copy(dsts.at[c * NS + s], dst_c)

    def _compact(v, off):
        dvec = dst_c[pl.ds(v * 16, 16)]
        svec = src_c[pl.ds(v * 16, 16)]
        m = dvec != NH
        plsc.store_compressed(dst_c.at[pl.ds(off, 16)], dvec, mask=m)
        plsc.store_compressed(src_c.at[pl.ds(off, 16)], svec, mask=m)
        return off + jnp.sum(jnp.where(m, 1, 0), axis=0)

    cnt = lax.fori_loop(0, EPP // 16, _compact, jnp.int32(0))
    dump_d = jnp.full((16,), NH, jnp.int32)
    dump_s = jnp.zeros((16,), jnp.int32)
    for t in range(2 * CW // 16):
        dst_c[pl.ds(cnt + 16 * t, 16)] = dump_d
        src_c[pl.ds(cnt + 16 * t, 16)] = dump_s
    nch = ((cnt + 2 * CW - 1) // (2 * CW)) * 2

    def g_start(tab, jj, b):
        pltpu.async_copy(tab.at[src_c.at[pl.ds(jj * CW, CW)]], rows.at[b],
                         sem_g.at[b])

    def g_wait(tab, jj, b):
        pltpu.make_async_copy(tab.at[src_c.at[pl.ds(jj * CW, CW)]], rows.at[b],
                              sem_g.at[b]).wait()

    def s_start(jj, b):
        for k in range(CW // 16):
            idxv = dst_c[pl.ds(jj * CW + 16 * k, 16)]
            pltpu.async_copy(rows.at[b].at[pl.ds(16 * k, 16)], acc.at[idxv],
                             sem_s.at[b], add=True)

    def s_wait(jj, b):
        pltpu.make_async_copy(h2n.at[pl.ds(0, CW)], rows.at[b],
                              sem_s.at[b]).wait()

    for p in range(2):
        tab = h2n.at[pl.ds(p * N, N)]
        zoff = jnp.minimum(s * SRT, AROWS - SRT)
        pltpu.sync_copy(zeros, acc.at[pl.ds(zoff, SRT)])
        plsc.subcore_barrier()

        @pl.when(nch > 0)
        def _():
            g_start(tab, 0, 0)

        @pl.when(nch > 1)
        def _():
            g_start(tab, 1, 1)

        @pl.loop(0, CH2, step=4)
        def _(j):
            for b in range(4):
                jj = j + b

                @pl.when(jj < nch)
                def _():
                    g_wait(tab, jj, b)
                    s_start(jj, b)

                @pl.when((jj >= 2) & (jj - 2 < nch))
                def _():
                    s_wait(jj - 2, (b + 2) % 4)

                @pl.when(jj + 2 < nch)
                def _():
                    g_start(tab, jj + 2, (b + 2) % 4)

        @pl.when(nch >= CH2 - 1)
        def _():
            s_wait(CH2 - 2, (CH2 - 2) % 4)

        @pl.when(nch == CH2)
        def _():
            s_wait(CH2 - 1, (CH2 - 1) % 4)

        plsc.subcore_barrier()
        woff = jnp.minimum(s * SRT, NH - SRT)
        pltpu.sync_copy(acc.at[pl.ds(woff, SRT)],
                        out.at[pl.ds(p * N + c * NH + woff, SRT)])
        plsc.subcore_barrier()


@functools.cache
def _build_seg_sum():
    return pl.kernel(
        _seg_body,
        out_type=jax.ShapeDtypeStruct((2 * N, D), jnp.float32),
        name="gin_segment_sum",
        mesh=plsc.VectorSubcoreMesh(core_axis_name="c", subcore_axis_name="s",
                                    num_cores=NC, num_subcores=NS),
        compiler_params=pltpu.CompilerParams(needs_layout_passes=False),
        scratch_types=[
            pltpu.VMEM((EPP,), jnp.int32),
            pltpu.VMEM((EPP,), jnp.int32),
            pltpu.VMEM((4, CW, D), jnp.float32),
            pltpu.VMEM_SHARED((AROWS, D), jnp.float32),
            pltpu.SemaphoreType.DMA((4,)),
            pltpu.SemaphoreType.DMA((4,)),
        ],
    )


def _seg_sum(h2n, srcs, dsts, zeros):
    return _build_seg_sum()(h2n, srcs, dsts, zeros)


def _stats_update(st_ref, yy, j):
    @pl.when(j == 0)
    def _():
        st_ref[...] = jnp.zeros_like(st_ref)

    st_ref[...] += jnp.stack(
        [jnp.sum(yy, axis=0), jnp.sum(yy * yy, axis=0)])[None]


def _bn(x, st, g, be):
    mean = st[0, 0:1, :] * (1.0 / N)
    var = st[0, 1:2, :] * (1.0 / N) - mean * mean
    inv = lax.rsqrt(var + 1e-5)
    return (x - mean) * (inv * g) + be


def _mlp_in_body(pooled_ref, h_ref, eps_ref, w_ref, b_ref, y_ref, st_ref):
    x = (pooled_ref[...].astype(jnp.float32)
         + (1.0 + eps_ref[0, 0]) * h_ref[...].astype(jnp.float32))
    yy = jnp.dot(x, w_ref[...], preferred_element_type=jnp.float32) + b_ref[...]
    y_ref[...] = yy
    _stats_update(st_ref, yy, pl.program_id(1))


def _mlp_mid_body(x_ref, st_ref, g_ref, be_ref, w_ref, b_ref, y_ref, sto_ref):
    xa = jnp.maximum(_bn(x_ref[...], st_ref[...], g_ref[...], be_ref[...]), 0.0)
    yy = jnp.dot(xa, w_ref[...], preferred_element_type=jnp.float32) + b_ref[...]
    y_ref[...] = yy
    _stats_update(sto_ref, yy, pl.program_id(1))


def _bnrelu_body(x_ref, st_ref, g_ref, be_ref, y_ref):
    y_ref[...] = jnp.maximum(
        _bn(x_ref[...], st_ref[...], g_ref[...], be_ref[...]), 0.0)


def _readout_body(x_ref, st_ref, g_ref, be_ref, m_ref, h_ref, csum_ref, cnt_ref):
    hh = jnp.maximum(_bn(x_ref[...], st_ref[...], g_ref[...], be_ref[...]), 0.0)
    h_ref[...] = hh
    p, j = pl.program_id(0), pl.program_id(1)

    @pl.when((p == 0) & (j == 0))
    def _():
        csum_ref[...] = jnp.zeros_like(csum_ref)
        cnt_ref[...] = jnp.zeros_like(cnt_ref)

    @pl.when(p == 0)
    def _():
        mm = m_ref[...]
        csum_ref[...] += lax.dot_general(
            hh, mm, (((0,), (0,)), ((), ())), preferred_element_type=jnp.float32)
        cnt_ref[...] += jnp.sum(mm, axis=0, keepdims=True)


def _loss_body(h_ref, m_ref, csum_ref, cnt_ref, wb_ref, bb_ref, acc_ref, loss_ref):
    p, j = pl.program_id(0), pl.program_id(1)
    cnt = cnt_ref[...]
    cm = csum_ref[...] / cnt
    summ = 1.0 / (1.0 + jnp.exp(-cm))
    s = jnp.dot(h_ref[...], wb_ref[...], preferred_element_type=jnp.float32)
    sc = jnp.dot(s, summ, preferred_element_type=jnp.float32) + bb_ref[0, 0]
    x = jnp.where(p == 0, -sc, sc)
    t = jnp.maximum(x, 0.0) + jnp.log(1.0 + jnp.exp(-jnp.abs(x)))
    contrib = jnp.sum(m_ref[...] * t, axis=0, keepdims=True)

    @pl.when((p == 0) & (j == 0))
    def _():
        acc_ref[...] = jnp.zeros_like(acc_ref)

    @pl.when(p == 0)
    def _():
        acc_ref[0:1, :] += contrib

    @pl.when(p == 1)
    def _():
        acc_ref[1:2, :] += contrib

    @pl.when((p == 1) & (j == NBLK - 1))
    def _():
        a = acc_ref[...]
        lk = (a[0:1, :] + a[1:2, :]) / (2.0 * cnt)
        mask = lax.broadcasted_iota(jnp.int32, (1, KP), 1) < K
        loss_ref[0, 0] = jnp.sum(jnp.where(mask, lk, 0.0)) * (1.0 / K)


_row_spec = pl.BlockSpec((BN, D), lambda p, j: (p * NBLK + j, 0))
_st_spec = pl.BlockSpec((1, 2, D), lambda p, j: (p, 0, 0))
_w_spec = pl.BlockSpec((D, D), lambda p, j: (0, 0))
_v_spec = pl.BlockSpec((1, D), lambda p, j: (0, 0))
_m_spec = pl.BlockSpec((BN, KP), lambda p, j: (j, 0))
_smem_spec = pl.BlockSpec(memory_space=pltpu.SMEM)

_x_shape = jax.ShapeDtypeStruct((2 * N, D), jnp.float32)
_st_shape = jax.ShapeDtypeStruct((2, 2, D), jnp.float32)


def _mlp_in(pooled, h, eps_l, w, b):
    return pl.pallas_call(
        _mlp_in_body, grid=(2, NBLK),
        in_specs=[_row_spec, _row_spec, _smem_spec, _w_spec, _v_spec],
        out_specs=[_row_spec, _st_spec],
        out_shape=[_x_shape, _st_shape],
    )(pooled, h, eps_l, w, b)


def _mlp_mid(x, st, g, be, w, b):
    return pl.pallas_call(
        _mlp_mid_body, grid=(2, NBLK),
        in_specs=[_row_spec, _st_spec, _v_spec, _v_spec, _w_spec, _v_spec],
        out_specs=[_row_spec, _st_spec],
        out_shape=[_x_shape, _st_shape],
    )(x, st, g, be, w, b)


def _bnrelu(x, st, g, be):
    return pl.pallas_call(
        _bnrelu_body, grid=(2, NBLK),
        in_specs=[_row_spec, _st_spec, _v_spec, _v_spec],
        out_specs=_row_spec,
        out_shape=_x_shape,
    )(x, st, g, be)


def _readout(x, st, g, be, m):
    return pl.pallas_call(
        _readout_body, grid=(2, NBLK),
        in_specs=[_row_spec, _st_spec, _v_spec, _v_spec, _m_spec],
        out_specs=[_row_spec,
                   pl.BlockSpec((D, KP), lambda p, j: (0, 0)),
                   pl.BlockSpec((1, KP), lambda p, j: (0, 0))],
        out_shape=[_x_shape,
                   jax.ShapeDtypeStruct((D, KP), jnp.float32),
                   jax.ShapeDtypeStruct((1, KP), jnp.float32)],
    )(x, st, g, be, m)


def _loss(h, m, csum, cnt, wb, bb):
    return pl.pallas_call(
        _loss_body, grid=(2, NBLK),
        in_specs=[_row_spec, _m_spec,
                  pl.BlockSpec((D, KP), lambda p, j: (0, 0)),
                  pl.BlockSpec((1, KP), lambda p, j: (0, 0)),
                  _w_spec, _smem_spec],
        out_specs=[pl.BlockSpec((2, KP), lambda p, j: (0, 0)), _smem_spec],
        out_shape=[jax.ShapeDtypeStruct((2, KP), jnp.float32),
                   jax.ShapeDtypeStruct((1, 1), jnp.float32)],
    )(h, m, csum, cnt, wb, bb)


def kernel(seq1, seq2, edge_index, loc, eps, W1s, b1s, g1s, be1s, W2s, b2s,
           g2s, be2s, Wb, bb):
    srcs = jnp.pad(edge_index[0].astype(jnp.int32).reshape(NS, EP),
                   ((0, 0), (0, EPP - EP)))
    dst = edge_index[1].astype(jnp.int32).reshape(NS, EP)
    dsts = jnp.pad(jnp.concatenate([jnp.where(dst < NH, dst, NH),
                                    jnp.where(dst >= NH, dst - NH, NH)],
                                   axis=0),
                   ((0, 0), (0, EPP - EP)), constant_values=NH)
    zeros = jnp.zeros((SRT, D), jnp.float32)
    locp = jnp.pad(loc.astype(jnp.float32), ((0, 0), (0, KP - K)))
    h = jnp.concatenate([seq1, seq2], axis=0)

    x2 = st2 = None
    for l in range(NLAYERS):
        pooled = _seg_sum(h, srcs, dsts, zeros)
        eps_l = eps[l].reshape(1, 1)
        x1, st1 = _mlp_in(pooled, h, eps_l, W1s[l], b1s[l].reshape(1, D))
        x2, st2 = _mlp_mid(x1, st1, g1s[l].reshape(1, D), be1s[l].reshape(1, D),
                           W2s[l], b2s[l].reshape(1, D))
        if l < NLAYERS - 1:
            h = _bnrelu(x2, st2, g2s[l].reshape(1, D), be2s[l].reshape(1, D))

    hf, csum, cnt = _readout(x2, st2, g2s[-1].reshape(1, D),
                             be2s[-1].reshape(1, D), locp)
    _, loss = _loss(hf, locp, csum, cnt, Wb, bb.reshape(1, 1))
    return loss.reshape(())


# speedup vs baseline: 4.7772x; 1.0186x over previous
import functools

import jax
import jax.numpy as jnp
from jax import lax
from jax.experimental import pallas as pl
from jax.experimental.pallas import tpu as pltpu
from jax.experimental.pallas import tpu_sc as plsc

N = 10000
D = 128
E = 320000
K = 10
KP = 128
NLAYERS = 2
NC = 2
NS = 16
EP = E // NS
CW = 80
CH2 = EP // CW + 2
EPP = CH2 * CW
NH = N // 2
AROWS = NH + 8
SRT = 320
BN = 5000
NBLK = N // BN


def _seg_body(h2n, srcs, dsts, zeros, out,
              src_c, dst_c, rows, acc, sem_g, sem_s):
    c = lax.axis_index("c")
    s = lax.axis_index("s")
    pltpu.sync_copy(srcs.at[s], src_c)
    pltpu.sync_copy(dsts.at[c * NS + s], dst_c)

    def _compact(v, off):
        dvec = dst_c[pl.ds(v * 16, 16)]
        svec = src_c[pl.ds(v * 16, 16)]
        m = dvec != NH
        plsc.store_compressed(dst_c.at[pl.ds(off, 16)], dvec, mask=m)
        plsc.store_compressed(src_c.at[pl.ds(off, 16)], svec, mask=m)
        return off + jnp.sum(jnp.where(m, 1, 0), axis=0)

    cnt = lax.fori_loop(0, EPP // 16, _compact, jnp.int32(0))
    dump_d = jnp.full((16,), NH, jnp.int32)
    dump_s = jnp.zeros((16,), jnp.int32)
    for t in range(2 * CW // 16):
        dst_c[pl.ds(cnt + 16 * t, 16)] = dump_d
        src_c[pl.ds(cnt + 16 * t, 16)] = dump_s
    nch = ((cnt + 2 * CW - 1) // (2 * CW)) * 2

    def g_start(tab, jj, b):
        pltpu.async_copy(tab.at[src_c.at[pl.ds(jj * CW, CW)]], rows.at[b],
                         sem_g.at[b])

    def g_wait(tab, jj, b):
        pltpu.make_async_copy(tab.at[src_c.at[pl.ds(jj * CW, CW)]], rows.at[b],
                              sem_g.at[b]).wait()

    def s_start(jj, b):
        for k in range(CW // 16):
            idxv = dst_c[pl.ds(jj * CW + 16 * k, 16)]
            pltpu.async_copy(rows.at[b].at[pl.ds(16 * k, 16)], acc.at[idxv],
                             sem_s.at[b], add=True)

    def s_wait(jj, b):
        pltpu.make_async_copy(h2n.at[pl.ds(0, CW)], rows.at[b],
                              sem_s.at[b]).wait()

    for p in range(2):
        tab = h2n.at[pl.ds(p * N, N)]
        zoff = jnp.minimum(s * SRT, AROWS - SRT)
        pltpu.sync_copy(zeros, acc.at[pl.ds(zoff, SRT)])
        plsc.subcore_barrier()

        @pl.when(nch > 0)
        def _():
            g_start(tab, 0, 0)

        @pl.when(nch > 1)
        def _():
            g_start(tab, 1, 1)

        @pl.loop(0, CH2, step=4)
        def _(j):
            for b in range(4):
                jj = j + b

                @pl.when(jj < nch)
                def _():
                    g_wait(tab, jj, b)
                    s_start(jj, b)

                @pl.when((jj >= 2) & (jj - 2 < nch))
                def _():
                    s_wait(jj - 2, (b + 2) % 4)

                @pl.when(jj + 2 < nch)
                def _():
                    g_start(tab, jj + 2, (b + 2) % 4)

        @pl.when(nch >= CH2 - 1)
        def _():
            s_wait(CH2 - 2, (CH2 - 2) % 4)

        @pl.when(nch == CH2)
        def _():
            s_wait(CH2 - 1, (CH2 - 1) % 4)

        plsc.subcore_barrier()
        woff = jnp.minimum(s * SRT, NH - SRT)
        pltpu.sync_copy(acc.at[pl.ds(woff, SRT)],
                        out.at[pl.ds(p * N + c * NH + woff, SRT)])
        plsc.subcore_barrier()


@functools.cache
def _build_seg_sum():
    return pl.kernel(
        _seg_body,
        out_type=jax.ShapeDtypeStruct((2 * N, D), jnp.float32),
        name="gin_segment_sum",
        mesh=plsc.VectorSubcoreMesh(core_axis_name="c", subcore_axis_name="s",
                                    num_cores=NC, num_subcores=NS),
        compiler_params=pltpu.CompilerParams(needs_layout_passes=False),
        scratch_types=[
            pltpu.VMEM((EPP,), jnp.int32),
            pltpu.VMEM((EPP,), jnp.int32),
            pltpu.VMEM((4, CW, D), jnp.float32),
            pltpu.VMEM_SHARED((AROWS, D), jnp.float32),
            pltpu.SemaphoreType.DMA((4,)),
            pltpu.SemaphoreType.DMA((4,)),
        ],
    )


def _seg_sum(h2n, srcs, dsts, zeros):
    return _build_seg_sum()(h2n, srcs, dsts, zeros)


def _stats_update(st_ref, yy, j):
    @pl.when(j == 0)
    def _():
        st_ref[...] = jnp.zeros_like(st_ref)

    st_ref[...] += jnp.stack(
        [jnp.sum(yy, axis=0), jnp.sum(yy * yy, axis=0)])[None]


def _bn(x, st, g, be):
    mean = st[0, 0:1, :] * (1.0 / N)
    var = st[0, 1:2, :] * (1.0 / N) - mean * mean
    inv = lax.rsqrt(var + 1e-5)
    return (x - mean) * (inv * g) + be


def _mlp_in_body(pooled_ref, h_ref, eps_ref, w_ref, b_ref, y_ref, st_ref):
    x = (pooled_ref[...].astype(jnp.float32)
         + (1.0 + eps_ref[0, 0]) * h_ref[...].astype(jnp.float32))
    yy = jnp.dot(x, w_ref[...], preferred_element_type=jnp.float32) + b_ref[...]
    y_ref[...] = yy
    _stats_update(st_ref, yy, pl.program_id(1))


def _mlp_mid_body(x_ref, st_ref, g_ref, be_ref, w_ref, b_ref, y_ref, sto_ref):
    xa = jnp.maximum(_bn(x_ref[...], st_ref[...], g_ref[...], be_ref[...]), 0.0)
    yy = jnp.dot(xa, w_ref[...], preferred_element_type=jnp.float32) + b_ref[...]
    y_ref[...] = yy
    _stats_update(sto_ref, yy, pl.program_id(1))


def _bnrelu_body(x_ref, st_ref, g_ref, be_ref, y_ref):
    y_ref[...] = jnp.maximum(
        _bn(x_ref[...], st_ref[...], g_ref[...], be_ref[...]), 0.0)


def _readout_body(x_ref, st_ref, g_ref, be_ref, m_ref, h_ref, csum_ref, cnt_ref):
    hh = jnp.maximum(_bn(x_ref[...], st_ref[...], g_ref[...], be_ref[...]), 0.0)
    h_ref[...] = hh
    p, j = pl.program_id(0), pl.program_id(1)

    @pl.when((p == 0) & (j == 0))
    def _():
        csum_ref[...] = jnp.zeros_like(csum_ref)
        cnt_ref[...] = jnp.zeros_like(cnt_ref)

    @pl.when(p == 0)
    def _():
        mm = m_ref[...]
        csum_ref[...] += lax.dot_general(
            hh, mm, (((0,), (0,)), ((), ())), preferred_element_type=jnp.float32)
        cnt_ref[...] += jnp.sum(mm, axis=0, keepdims=True)


def _loss_body(h_ref, m_ref, csum_ref, cnt_ref, wb_ref, bb_ref, acc_ref, loss_ref):
    p, j = pl.program_id(0), pl.program_id(1)
    cnt = cnt_ref[...]
    cm = csum_ref[...] / cnt
    summ = 1.0 / (1.0 + jnp.exp(-cm))
    s = jnp.dot(h_ref[...], wb_ref[...], preferred_element_type=jnp.float32)
    sc = jnp.dot(s, summ, preferred_element_type=jnp.float32) + bb_ref[0, 0]
    x = jnp.where(p == 0, -sc, sc)
    t = jnp.maximum(x, 0.0) + jnp.log(1.0 + jnp.exp(-jnp.abs(x)))
    contrib = jnp.sum(m_ref[...] * t, axis=0, keepdims=True)

    @pl.when((p == 0) & (j == 0))
    def _():
        acc_ref[...] = jnp.zeros_like(acc_ref)

    @pl.when(p == 0)
    def _():
        acc_ref[0:1, :] += contrib

    @pl.when(p == 1)
    def _():
        acc_ref[1:2, :] += contrib

    @pl.when((p == 1) & (j == NBLK - 1))
    def _():
        a = acc_ref[...]
        lk = (a[0:1, :] + a[1:2, :]) / (2.0 * cnt)
        mask = lax.broadcasted_iota(jnp.int32, (1, KP), 1) < K
        loss_ref[0, 0] = jnp.sum(jnp.where(mask, lk, 0.0)) * (1.0 / K)


_row_spec = pl.BlockSpec((BN, D), lambda p, j: (p * NBLK + j, 0))
_st_spec = pl.BlockSpec((1, 2, D), lambda p, j: (p, 0, 0))
_w_spec = pl.BlockSpec((D, D), lambda p, j: (0, 0))
_v_spec = pl.BlockSpec((1, D), lambda p, j: (0, 0))
_m_spec = pl.BlockSpec((BN, KP), lambda p, j: (j, 0))
_smem_spec = pl.BlockSpec(memory_space=pltpu.SMEM)

_x_shape = jax.ShapeDtypeStruct((2 * N, D), jnp.float32)
_st_shape = jax.ShapeDtypeStruct((2, 2, D), jnp.float32)


def _mlp_in(pooled, h, eps_l, w, b):
    return pl.pallas_call(
        _mlp_in_body, grid=(2, NBLK),
        in_specs=[_row_spec, _row_spec, _smem_spec, _w_spec, _v_spec],
        out_specs=[_row_spec, _st_spec],
        out_shape=[_x_shape, _st_shape],
    )(pooled, h, eps_l, w, b)


def _mlp_mid(x, st, g, be, w, b):
    return pl.pallas_call(
        _mlp_mid_body, grid=(2, NBLK),
        in_specs=[_row_spec, _st_spec, _v_spec, _v_spec, _w_spec, _v_spec],
        out_specs=[_row_spec, _st_spec],
        out_shape=[_x_shape, _st_shape],
    )(x, st, g, be, w, b)


def _bnrelu(x, st, g, be):
    return pl.pallas_call(
        _bnrelu_body, grid=(2, NBLK),
        in_specs=[_row_spec, _st_spec, _v_spec, _v_spec],
        out_specs=_row_spec,
        out_shape=_x_shape,
    )(x, st, g, be)


def _readout(x, st, g, be, m):
    return pl.pallas_call(
        _readout_body, grid=(2, NBLK),
        in_specs=[_row_spec, _st_spec, _v_spec, _v_spec, _m_spec],
        out_specs=[_row_spec,
                   pl.BlockSpec((D, KP), lambda p, j: (0, 0)),
                   pl.BlockSpec((1, KP), lambda p, j: (0, 0))],
        out_shape=[_x_shape,
                   jax.ShapeDtypeStruct((D, KP), jnp.float32),
                   jax.ShapeDtypeStruct((1, KP), jnp.float32)],
    )(x, st, g, be, m)


def _loss(h, m, csum, cnt, wb, bb):
    return pl.pallas_call(
        _loss_body, grid=(2, NBLK),
        in_specs=[_row_spec, _m_spec,
                  pl.BlockSpec((D, KP), lambda p, j: (0, 0)),
                  pl.BlockSpec((1, KP), lambda p, j: (0, 0)),
                  _w_spec, _smem_spec],
        out_specs=[pl.BlockSpec((2, KP), lambda p, j: (0, 0)), _smem_spec],
        out_shape=[jax.ShapeDtypeStruct((2, KP), jnp.float32),
                   jax.ShapeDtypeStruct((1, 1), jnp.float32)],
    )(h, m, csum, cnt, wb, bb)


def kernel(seq1, seq2, edge_index, loc, eps, W1s, b1s, g1s, be1s, W2s, b2s,
           g2s, be2s, Wb, bb):
    srcs = jnp.pad(edge_index[0].astype(jnp.int32).reshape(NS, EP),
                   ((0, 0), (0, EPP - EP)))
    dst = edge_index[1].astype(jnp.int32).reshape(NS, EP)
    dsts = jnp.pad(jnp.concatenate([jnp.where(dst < NH, dst, NH),
                                    jnp.where(dst >= NH, dst - NH, NH)],
                                   axis=0),
                   ((0, 0), (0, EPP - EP)), constant_values=NH)
    zeros = jnp.zeros((SRT, D), jnp.float32)
    locp = jnp.pad(loc.astype(jnp.float32), ((0, 0), (0, KP - K)))
    h = jnp.concatenate([seq1, seq2], axis=0)

    x2 = st2 = None
    for l in range(NLAYERS):
        pooled = _seg_sum(h, srcs, dsts, zeros)
        eps_l = eps[l].reshape(1, 1)
        x1, st1 = _mlp_in(pooled, h, eps_l, W1s[l], b1s[l].reshape(1, D))
        x2, st2 = _mlp_mid(x1, st1, g1s[l].reshape(1, D), be1s[l].reshape(1, D),
                           W2s[l], b2s[l].reshape(1, D))
        if l < NLAYERS - 1:
            h = _bnrelu(x2, st2, g2s[l].reshape(1, D), be2s[l].reshape(1, D))

    hf, csum, cnt = _readout(x2, st2, g2s[-1].reshape(1, D),
                             be2s[-1].reshape(1, D), locp)
    _, loss = _loss(hf, locp, csum, cnt, Wb, bb.reshape(1, 1))
    return loss.reshape(())
